# baseline (speedup 1.0000x reference)
import functools

import jax
import jax.numpy as jnp
from jax import lax
from jax.experimental import pallas as pl
from jax.experimental.pallas import tpu as pltpu
from jax.experimental.pallas import tpu_sc as plsc

_D = 64
_G = 128
_K = 4
_NC = 2
_NS = 16
_NW = _NC * _NS


def _make_kernel(n_idx):
    rows_w = n_idx // _NW
    gpw = rows_w // _G
    n_ch = gpw // _K
    mesh = plsc.VectorSubcoreMesh(core_axis_name="c", subcore_axis_name="s")

    @functools.partial(
        pl.kernel,
        mesh=mesh,
        compiler_params=pltpu.CompilerParams(use_tc_tiling_on_sc=False),
        out_type=jax.ShapeDtypeStruct((n_idx // _G, _G, _D), jnp.float32),
        scratch_types=[
            pltpu.VMEM((gpw, _G), jnp.int32),
            pltpu.VMEM((_K, _G, _D), jnp.float32),
            pltpu.SemaphoreType.DMA,
        ],
    )
    def emb(idx_hbm, table_hbm, out_hbm, idx_v, rows_v, gsem):
        wid = lax.axis_index("s") * _NC + lax.axis_index("c")
        gbase = wid * gpw
        pltpu.sync_copy(idx_hbm.at[pl.ds(gbase, gpw)], idx_v)

        def chunk(ci, carry):
            cps = [
                pltpu.async_copy(
                    table_hbm.at[idx_v.at[ci * _K + b]], rows_v.at[b], gsem)
                for b in range(_K)
            ]
            for cp in cps:
                cp.wait()
            pltpu.sync_copy(rows_v, out_hbm.at[pl.ds(gbase + ci * _K, _K)])
            return carry

        lax.fori_loop(0, n_ch, chunk, 0)

    return emb


def kernel(token_ids, embeddings):
    b, s = token_ids.shape
    n = b * s
    idx = token_ids.reshape(n // _G, _G).astype(jnp.int32)
    out = _make_kernel(n)(idx, embeddings)
    return out.reshape(b, s, _D)
